# Initial kernel scaffold; baseline (speedup 1.0000x reference)
import jax
import jax.numpy as jnp
from jax.experimental import pallas as pl


def kernel(features):
    raise NotImplementedError("write your pallas kernel here")


# speedup vs baseline: 11.6117x; 11.6117x over previous
import jax
import jax.numpy as jnp
from jax.experimental import pallas as pl
from jax.experimental.pallas import tpu as pltpu

_K = 32
_LAMBDA = 0.1
_TEMP = 0.1
_NEG = -3.0e38
_R = 256


def _phase1_body(f_ref, fb_ref, sim_ref, a_ref, s_scr):
    f = f_ref[...]
    B = f.shape[0]
    n = jnp.sqrt(jnp.sum(f * f, axis=1, keepdims=True))
    x = f / jnp.maximum(n, 1e-12)
    fb = fb_ref[...]
    nb = jnp.sqrt(jnp.sum(fb * fb, axis=1, keepdims=True))
    xb = fb / jnp.maximum(nb, 1e-12)
    sim = jax.lax.dot_general(
        xb, x, (((1,), (1,)), ((), ())), preferred_element_type=jnp.float32)
    sim_ref[...] = sim
    iota = jax.lax.broadcasted_iota(jnp.int32, (_R, B), 1)

    m = jnp.max(sim, axis=1, keepdims=True)
    idx = jnp.min(jnp.where(sim >= m, iota, B), axis=1, keepdims=True)
    s_scr[...] = jnp.where(iota == idx, _NEG, sim)
    a_ref[...] = jnp.zeros((_R, B), jnp.float32)

    def body(t, carry):
        s = s_scr[...]
        m = jnp.max(s, axis=1, keepdims=True)
        idx = jnp.min(jnp.where(s >= m, iota, B), axis=1, keepdims=True)
        onehot = iota == idx
        s_scr[...] = jnp.where(onehot, _NEG, s)
        a_ref[...] += onehot.astype(jnp.float32)
        return carry

    jax.lax.fori_loop(0, _K, body, 0)


def _spmm_body(a_row_ref, a_col_ref, v_ref, out_ref):
    v = v_ref[...]
    acc = jax.lax.dot_general(
        a_row_ref[...], v, (((1,), (0,)), ((), ())),
        preferred_element_type=jnp.float32)
    acc = acc + jax.lax.dot_general(
        a_col_ref[...], v, (((0,), (0,)), ((), ())),
        preferred_element_type=jnp.float32)
    out_ref[...] = acc * (1.0 / _K)


def _phase3_body(sim_ref, fb_ref, f_ref, g_ref, out_ref):
    f = f_ref[...]
    g = g_ref[...]
    fb = fb_ref[...]
    gn = g / jnp.maximum(
        jnp.sqrt(jnp.sum(g * g, axis=1, keepdims=True)), 1e-8)
    fnb = fb / jnp.maximum(
        jnp.sqrt(jnp.sum(fb * fb, axis=1, keepdims=True)), 1e-8)
    cos = jax.lax.dot_general(
        fnb, gn, (((1,), (1,)), ((), ())), preferred_element_type=jnp.float32)
    logits = (sim_ref[...] + _LAMBDA * cos) / _TEMP
    m = jnp.max(logits, axis=1, keepdims=True)
    e = jnp.exp(logits - m)
    s = jnp.sum(e, axis=1, keepdims=True)
    acc = jax.lax.dot_general(
        e, f, (((1,), (0,)), ((), ())), preferred_element_type=jnp.float32)
    out_ref[...] = acc / s


def kernel(features):
    B, D = features.shape
    nblk = B // _R
    f32 = jnp.float32

    sim, a = pl.pallas_call(
        _phase1_body,
        grid=(nblk,),
        in_specs=[
            pl.BlockSpec((B, D), lambda i: (0, 0)),
            pl.BlockSpec((_R, D), lambda i: (i, 0)),
        ],
        out_specs=[
            pl.BlockSpec((_R, B), lambda i: (i, 0)),
            pl.BlockSpec((_R, B), lambda i: (i, 0)),
        ],
        out_shape=[
            jax.ShapeDtypeStruct((B, B), f32),
            jax.ShapeDtypeStruct((B, B), f32),
        ],
        scratch_shapes=[pltpu.VMEM((_R, B), f32)],
    )(features, features)

    def spmm(v):
        return pl.pallas_call(
            _spmm_body,
            grid=(nblk,),
            in_specs=[
                pl.BlockSpec((_R, B), lambda i: (i, 0)),
                pl.BlockSpec((B, _R), lambda i: (0, i)),
                pl.BlockSpec((B, D), lambda i: (0, 0)),
            ],
            out_specs=pl.BlockSpec((_R, D), lambda i: (i, 0)),
            out_shape=jax.ShapeDtypeStruct((B, D), f32),
        )(a, a, v)

    diff1 = spmm(features)
    geo = spmm(diff1)

    enhanced = pl.pallas_call(
        _phase3_body,
        grid=(nblk,),
        in_specs=[
            pl.BlockSpec((_R, B), lambda i: (i, 0)),
            pl.BlockSpec((_R, D), lambda i: (i, 0)),
            pl.BlockSpec((B, D), lambda i: (0, 0)),
            pl.BlockSpec((B, D), lambda i: (0, 0)),
        ],
        out_specs=pl.BlockSpec((_R, D), lambda i: (i, 0)),
        out_shape=jax.ShapeDtypeStruct((B, D), f32),
    )(sim, features, features, geo)

    return enhanced


# speedup vs baseline: 21.2190x; 1.8274x over previous
import jax
import jax.numpy as jnp
from jax.experimental import pallas as pl
from jax.experimental.pallas import tpu as pltpu

_K = 32
_LAMBDA = 0.1
_TEMP = 0.1
_NEG = -3.0e38
_R = 256


def _phase1_body(f_ref, fb_ref, sim_ref, a_ref):
    f = f_ref[...]
    B = f.shape[0]
    n = jnp.sqrt(jnp.sum(f * f, axis=1, keepdims=True))
    x = f / jnp.maximum(n, 1e-12)
    fb = fb_ref[...]
    nb = jnp.sqrt(jnp.sum(fb * fb, axis=1, keepdims=True))
    xb = fb / jnp.maximum(nb, 1e-12)
    sim = jax.lax.dot_general(
        xb, x, (((1,), (1,)), ((), ())), preferred_element_type=jnp.float32)
    sim_ref[...] = sim
    iota = jax.lax.broadcasted_iota(jnp.int32, (_R, B), 1)

    bits = jax.lax.bitcast_convert_type(sim, jnp.uint32)
    ukey = jnp.where(
        (bits >> 31) != 0, ~bits, bits | jnp.uint32(0x80000000))

    kk = _K + 1

    def bit_round(t, prefix):
        cand = prefix | (jnp.uint32(1) << (jnp.uint32(31) - t.astype(jnp.uint32)))
        cnt = jnp.sum((ukey >= cand).astype(jnp.int32), axis=1, keepdims=True)
        return jnp.where(cnt >= kk, cand, prefix)

    t33 = jax.lax.fori_loop(
        0, 32, bit_round, jnp.zeros((_R, 1), jnp.uint32))

    eq = ukey == t33
    cnt_gt = jnp.sum((ukey > t33).astype(jnp.int32), axis=1, keepdims=True)
    need = kk - cnt_gt
    w = jnp.where(eq, B - iota, 0)

    def idx_round(t, prefix):
        cand = prefix | (jnp.int32(1) << (jnp.int32(12) - t))
        cnt = jnp.sum((w >= cand).astype(jnp.int32), axis=1, keepdims=True)
        return jnp.where(cnt >= need, cand, prefix)

    wstar = jax.lax.fori_loop(
        0, 13, idx_round, jnp.zeros((_R, 1), jnp.int32))
    sel = (ukey > t33) | (w >= wstar)

    m = jnp.max(sim, axis=1, keepdims=True)
    i0 = jnp.min(jnp.where(sim == m, iota, B), axis=1, keepdims=True)
    sel = sel & (iota != i0)
    a_ref[...] = jnp.where(sel, 1.0, 0.0).astype(jnp.float32)


def _spmm_body(a_row_ref, a_col_ref, v_ref, out_ref):
    v = v_ref[...]
    acc = jax.lax.dot_general(
        a_row_ref[...], v, (((1,), (0,)), ((), ())),
        preferred_element_type=jnp.float32)
    acc = acc + jax.lax.dot_general(
        a_col_ref[...], v, (((0,), (0,)), ((), ())),
        preferred_element_type=jnp.float32)
    out_ref[...] = acc * (1.0 / _K)


def _phase3_body(sim_ref, fb_ref, f_ref, g_ref, out_ref):
    f = f_ref[...]
    g = g_ref[...]
    fb = fb_ref[...]
    gn = g / jnp.maximum(
        jnp.sqrt(jnp.sum(g * g, axis=1, keepdims=True)), 1e-8)
    fnb = fb / jnp.maximum(
        jnp.sqrt(jnp.sum(fb * fb, axis=1, keepdims=True)), 1e-8)
    cos = jax.lax.dot_general(
        fnb, gn, (((1,), (1,)), ((), ())), preferred_element_type=jnp.float32)
    logits = (sim_ref[...] + _LAMBDA * cos) / _TEMP
    m = jnp.max(logits, axis=1, keepdims=True)
    e = jnp.exp(logits - m)
    s = jnp.sum(e, axis=1, keepdims=True)
    acc = jax.lax.dot_general(
        e, f, (((1,), (0,)), ((), ())), preferred_element_type=jnp.float32)
    out_ref[...] = acc / s


def kernel(features):
    B, D = features.shape
    nblk = B // _R
    f32 = jnp.float32

    sim, a = pl.pallas_call(
        _phase1_body,
        grid=(nblk,),
        in_specs=[
            pl.BlockSpec((B, D), lambda i: (0, 0)),
            pl.BlockSpec((_R, D), lambda i: (i, 0)),
        ],
        out_specs=[
            pl.BlockSpec((_R, B), lambda i: (i, 0)),
            pl.BlockSpec((_R, B), lambda i: (i, 0)),
        ],
        out_shape=[
            jax.ShapeDtypeStruct((B, B), f32),
            jax.ShapeDtypeStruct((B, B), f32),
        ],
    )(features, features)

    def spmm(v):
        return pl.pallas_call(
            _spmm_body,
            grid=(nblk,),
            in_specs=[
                pl.BlockSpec((_R, B), lambda i: (i, 0)),
                pl.BlockSpec((B, _R), lambda i: (0, i)),
                pl.BlockSpec((B, D), lambda i: (0, 0)),
            ],
            out_specs=pl.BlockSpec((_R, D), lambda i: (i, 0)),
            out_shape=jax.ShapeDtypeStruct((B, D), f32),
        )(a, a, v)

    diff1 = spmm(features)
    geo = spmm(diff1)

    enhanced = pl.pallas_call(
        _phase3_body,
        grid=(nblk,),
        in_specs=[
            pl.BlockSpec((_R, B), lambda i: (i, 0)),
            pl.BlockSpec((_R, D), lambda i: (i, 0)),
            pl.BlockSpec((B, D), lambda i: (0, 0)),
            pl.BlockSpec((B, D), lambda i: (0, 0)),
        ],
        out_specs=pl.BlockSpec((_R, D), lambda i: (i, 0)),
        out_shape=jax.ShapeDtypeStruct((B, D), f32),
    )(sim, features, features, geo)

    return enhanced
